# baseline (speedup 1.0000x reference)
import jax
import jax.numpy as jnp
from jax import lax
from jax.experimental import pallas as pl
from jax.experimental.pallas import tpu as pltpu
from jax.experimental.pallas import tpu_sc as plsc

_DIM = 1024
_SCALE = _DIM ** (-0.5)
_ROWS = 8192
_TOTAL = _ROWS * _DIM
_NC = 2
_NS = 16
_NW = _NC * _NS
_WELEMS = _TOTAL // _NW
_CHUNK_E = 32 * _DIM
_NCHUNK = _WELEMS // _CHUNK_E


def _sc_body(emb_hbm, out_hbm, buf0, buf1, ls0, ls1, ss0, ss1):
    c = lax.axis_index("c")
    s = lax.axis_index("s")
    base = (s * _NC + c) * _WELEMS
    bufs = (buf0, buf1)
    lsems = (ls0, ls1)
    ssems = (ss0, ss1)

    pltpu.async_copy(emb_hbm.at[pl.ds(base, _CHUNK_E)], buf0, ls0)
    for g in range(_NCHUNK):
        b = g % 2
        off = base + g * _CHUNK_E
        pltpu.make_async_copy(
            emb_hbm.at[pl.ds(off, _CHUNK_E)], bufs[b], lsems[b]).wait()
        if g + 1 < _NCHUNK:
            nb = (g + 1) % 2
            if g >= 1:
                pltpu.make_async_copy(
                    bufs[nb],
                    out_hbm.at[pl.ds(off - _CHUNK_E, _CHUNK_E)],
                    ssems[nb]).wait()
            pltpu.async_copy(
                emb_hbm.at[pl.ds(off + _CHUNK_E, _CHUNK_E)],
                bufs[nb], lsems[nb])

        @plsc.parallel_loop(0, _CHUNK_E, 16, unroll=8)
        def _scale(i):
            idx = pl.multiple_of(i, 16)
            bufs[b][pl.ds(idx, 16)] = bufs[b][pl.ds(idx, 16)] * _SCALE

        pltpu.async_copy(bufs[b], out_hbm.at[pl.ds(off, _CHUNK_E)], ssems[b])

    pltpu.make_async_copy(
        bufs[0], out_hbm.at[pl.ds(base, _CHUNK_E)], ssems[0]).wait()
    pltpu.make_async_copy(
        bufs[1], out_hbm.at[pl.ds(base, _CHUNK_E)], ssems[1]).wait()


_sc_scale = pl.kernel(
    _sc_body,
    out_type=jax.ShapeDtypeStruct((_TOTAL,), jnp.float32),
    mesh=plsc.VectorSubcoreMesh(core_axis_name="c", subcore_axis_name="s"),
    scratch_types=[
        pltpu.VMEM((_CHUNK_E,), jnp.float32),
        pltpu.VMEM((_CHUNK_E,), jnp.float32),
        pltpu.SemaphoreType.DMA,
        pltpu.SemaphoreType.DMA,
        pltpu.SemaphoreType.DMA,
        pltpu.SemaphoreType.DMA,
    ],
)


def kernel(x, emb):
    seq_len = x.shape[1]
    out_flat = _sc_scale(emb[:seq_len].reshape(-1))
    return out_flat.reshape(seq_len, _DIM)


# speedup vs baseline: 2.3921x; 2.3921x over previous
import jax
import jax.numpy as jnp
from jax import lax
from jax.experimental import pallas as pl
from jax.experimental.pallas import tpu as pltpu
from jax.experimental.pallas import tpu_sc as plsc

_DIM = 1024
_SCALE = _DIM ** (-0.5)
_ROWS = 8192
_NC = 2
_NS = 16
_NW = _NC * _NS
_WROWS = _ROWS // _NW
_CROWS = 32
_NCHUNK = _WROWS // _CROWS
_CVECS = _CROWS * _DIM // 16


def _sc_body(emb_hbm, out_hbm, buf0, buf1, ls0, ls1, ss0, ss1):
    c = lax.axis_index("c")
    s = lax.axis_index("s")
    base = (s * _NC + c) * _WROWS
    bufs = (buf0, buf1)
    lsems = (ls0, ls1)
    ssems = (ss0, ss1)

    pltpu.async_copy(emb_hbm.at[pl.ds(base, _CROWS), :], buf0, ls0)
    for g in range(_NCHUNK):
        b = g % 2
        row = base + g * _CROWS
        pltpu.make_async_copy(
            emb_hbm.at[pl.ds(row, _CROWS), :], bufs[b], lsems[b]).wait()
        if g + 1 < _NCHUNK:
            nb = (g + 1) % 2
            if g >= 1:
                pltpu.make_async_copy(
                    bufs[nb],
                    out_hbm.at[pl.ds(row - _CROWS, _CROWS), :],
                    ssems[nb]).wait()
            pltpu.async_copy(
                emb_hbm.at[pl.ds(row + _CROWS, _CROWS), :],
                bufs[nb], lsems[nb])

        @plsc.parallel_loop(0, _CVECS, 1, unroll=8)
        def _scale(i):
            r = i >> 6
            col = pl.multiple_of((i & 63) * 16, 16)
            bufs[b][r, pl.ds(col, 16)] = bufs[b][r, pl.ds(col, 16)] * _SCALE

        pltpu.async_copy(
            bufs[b], out_hbm.at[pl.ds(row, _CROWS), :], ssems[b])

    pltpu.make_async_copy(
        bufs[0], out_hbm.at[pl.ds(base, _CROWS), :], ssems[0]).wait()
    pltpu.make_async_copy(
        bufs[1], out_hbm.at[pl.ds(base, _CROWS), :], ssems[1]).wait()


_sc_scale = pl.kernel(
    _sc_body,
    out_type=jax.ShapeDtypeStruct((_ROWS, _DIM), jnp.float32),
    mesh=plsc.VectorSubcoreMesh(core_axis_name="c", subcore_axis_name="s"),
    scratch_types=[
        pltpu.VMEM((_CROWS, _DIM), jnp.float32),
        pltpu.VMEM((_CROWS, _DIM), jnp.float32),
        pltpu.SemaphoreType.DMA,
        pltpu.SemaphoreType.DMA,
        pltpu.SemaphoreType.DMA,
        pltpu.SemaphoreType.DMA,
    ],
)


def kernel(x, emb):
    seq_len = x.shape[1]
    return _sc_scale(emb[:seq_len])
